# Initial kernel scaffold; baseline (speedup 1.0000x reference)
import jax
import jax.numpy as jnp
from jax.experimental import pallas as pl


def kernel(x, edge_index):
    raise NotImplementedError("write your pallas kernel here")


# speedup vs baseline: 5.4528x; 5.4528x over previous
import functools

import jax
import jax.numpy as jnp
from jax import lax
from jax.experimental import pallas as pl
from jax.experimental.pallas import tpu as pltpu
from jax.experimental.pallas import tpu_sc as plsc

N_NODES = 10000
N_EDGES = 320000
D_FEAT = 128

NC = 2
NS = 16
NW = NC * NS

EPW = N_EDGES // NW
CHUNK = 80
NCHUNKS = EPW // CHUNK

N_PAD = 10240
ZROWS = N_PAD // NS
OROWS = 624
OTAIL = N_NODES - OROWS * NS


def _sc_scatter_gather(x, src, dst):
  mesh = plsc.VectorSubcoreMesh(core_axis_name="c", subcore_axis_name="s")

  @functools.partial(
      pl.kernel,
      out_type=jax.ShapeDtypeStruct((NC, N_NODES, D_FEAT), jnp.float32),
      mesh=mesh,
      scratch_types=[
          pltpu.VMEM((CHUNK,), jnp.int32),
          pltpu.VMEM((CHUNK,), jnp.int32),
          pltpu.VMEM((CHUNK, D_FEAT), jnp.float32),
          pltpu.VMEM_SHARED((N_PAD, D_FEAT), jnp.float32),
          pltpu.SemaphoreType.DMA,
      ],
  )
  def k(x_hbm, src_hbm, dst_hbm, out_hbm, src_v, dst_v, rows_v, acc, sem):
    cid = lax.axis_index("c")
    sid = lax.axis_index("s")
    tile = cid * NS + sid
    base = tile * EPW

    def zrow(i, _):
      for j in range(D_FEAT // 16):
        rows_v[i, pl.ds(j * 16, 16)] = jnp.zeros((16,), jnp.float32)
      return 0
    lax.fori_loop(0, CHUNK, zrow, 0)

    def zacc(z, _):
      pltpu.sync_copy(rows_v, acc.at[pl.ds(sid * ZROWS + z * CHUNK, CHUNK)])
      return 0
    lax.fori_loop(0, ZROWS // CHUNK, zacc, 0)
    plsc.subcore_barrier()

    def chunk(ci, _):
      off = base + ci * CHUNK
      pltpu.sync_copy(src_hbm.at[pl.ds(off, CHUNK)], src_v)
      pltpu.sync_copy(dst_hbm.at[pl.ds(off, CHUNK)], dst_v)
      pltpu.async_copy(x_hbm.at[src_v], rows_v, sem).wait()
      pltpu.sync_copy(rows_v, acc.at[dst_v], add=True)
      return 0
    lax.fori_loop(0, NCHUNKS, chunk, 0)

    plsc.subcore_barrier()
    pltpu.sync_copy(acc.at[pl.ds(sid * OROWS, OROWS)],
                    out_hbm.at[cid, pl.ds(sid * OROWS, OROWS)])

    @pl.when(sid == NS - 1)
    def _tail():
      pltpu.sync_copy(acc.at[pl.ds(OROWS * NS, OTAIL)],
                      out_hbm.at[cid, pl.ds(OROWS * NS, OTAIL)])

  return k(x, src, dst)


def _tc_add(a, b):
  def body(a_ref, b_ref, o_ref):
    o_ref[...] = a_ref[...] + b_ref[...]

  blk = 1000
  return pl.pallas_call(
      body,
      out_shape=jax.ShapeDtypeStruct((N_NODES, D_FEAT), jnp.float32),
      grid=(N_NODES // blk,),
      in_specs=[
          pl.BlockSpec((blk, D_FEAT), lambda i: (i, 0)),
          pl.BlockSpec((blk, D_FEAT), lambda i: (i, 0)),
      ],
      out_specs=pl.BlockSpec((blk, D_FEAT), lambda i: (i, 0)),
  )(a, b)


@jax.jit
def kernel(x, edge_index):
  src = edge_index[0]
  dst = edge_index[1]
  partial = _sc_scatter_gather(x, src, dst)
  return _tc_add(partial[0], partial[1])


# speedup vs baseline: 13.5243x; 2.4802x over previous
import functools

import jax
import jax.numpy as jnp
from jax import lax
from jax.experimental import pallas as pl
from jax.experimental.pallas import tpu as pltpu
from jax.experimental.pallas import tpu_sc as plsc

N_NODES = 10000
N_EDGES = 320000
D_FEAT = 128

NC = 2
NS = 16
NW = NC * NS

EPW = N_EDGES // NW
CHUNK = 80
NCHUNKS = EPW // CHUNK
NBUF = 3
NMAIN = NCHUNKS - (NCHUNKS % NBUF)

ZN = 8
OROWS = 624
OTAIL = N_NODES - OROWS * NS


def _sc_scatter_gather(x, src3, dst3):
  mesh = plsc.VectorSubcoreMesh(core_axis_name="c", subcore_axis_name="s")

  @functools.partial(
      pl.kernel,
      out_type=jax.ShapeDtypeStruct((NC, N_NODES, D_FEAT), jnp.float32),
      mesh=mesh,
      scratch_types=[
          pltpu.VMEM((NCHUNKS, CHUNK), jnp.int32),
          [pltpu.VMEM((CHUNK,), jnp.int32)] * NBUF,
          [pltpu.VMEM((CHUNK, D_FEAT), jnp.float32)] * NBUF,
          pltpu.VMEM_SHARED((N_NODES, D_FEAT), jnp.float32),
          [pltpu.SemaphoreType.DMA] * NBUF,
          [pltpu.SemaphoreType.DMA] * NBUF,
          [pltpu.SemaphoreType.DMA] * NBUF,
      ],
  )
  def k(x_hbm, src_hbm, dst_hbm, out_hbm,
        srcs, dstv, rows, acc, gsem, dsem, ssem):
    cid = lax.axis_index("c")
    sid = lax.axis_index("s")
    tile = cid * NS + sid

    pltpu.sync_copy(src_hbm.at[tile], srcs)
    ebase = tile * EPW

    @pl.loop(0, CHUNK)
    def _zrow(i):
      for j in range(D_FEAT // 16):
        rows[0][i, pl.ds(j * 16, 16)] = jnp.zeros((16,), jnp.float32)

    @pl.loop(0, ZN)
    def _zacc(z):
      pltpu.sync_copy(rows[0], acc.at[pl.ds(sid * OROWS + z * CHUNK, CHUNK)])

    plsc.subcore_barrier()

    for b in range(NBUF):
      pltpu.async_copy(dst_hbm.at[pl.ds(ebase + b * CHUNK, CHUNK)],
                       dstv[b], dsem[b])
      pltpu.async_copy(x_hbm.at[srcs.at[b]], rows[b], gsem[b])

    def _do_chunk(ci, b):
      pltpu.make_async_copy(x_hbm.at[srcs.at[ci]], rows[b], gsem[b]).wait()
      pltpu.make_async_copy(dst_hbm.at[pl.ds(0, CHUNK)], dstv[b],
                            dsem[b]).wait()
      pltpu.async_copy(rows[b], acc.at[dstv[b]], ssem[b], add=True).wait()

    @pl.loop(0, NMAIN, step=NBUF)
    def _group(ci0):
      for b in range(NBUF):
        ci = ci0 + b
        _do_chunk(ci, b)

        @pl.when(ci + NBUF < NCHUNKS)
        def _prefetch():
          pltpu.async_copy(
              dst_hbm.at[pl.ds(ebase + (ci + NBUF) * CHUNK, CHUNK)],
              dstv[b], dsem[b])
          pltpu.async_copy(x_hbm.at[srcs.at[ci + NBUF]], rows[b], gsem[b])

    for ci in range(NMAIN, NCHUNKS):
      _do_chunk(ci, ci % NBUF)

    plsc.subcore_barrier()
    pltpu.sync_copy(acc.at[pl.ds(sid * OROWS, OROWS)],
                    out_hbm.at[cid, pl.ds(sid * OROWS, OROWS)])

    @pl.when(sid == NS - 1)
    def _tail():
      pltpu.sync_copy(acc.at[pl.ds(OROWS * NS, OTAIL)],
                      out_hbm.at[cid, pl.ds(OROWS * NS, OTAIL)])

  return k(x, src3, dst3)


def _tc_add(a, b):
  def body(a_ref, b_ref, o_ref):
    o_ref[...] = a_ref[...] + b_ref[...]

  blk = 1000
  return pl.pallas_call(
      body,
      out_shape=jax.ShapeDtypeStruct((N_NODES, D_FEAT), jnp.float32),
      grid=(N_NODES // blk,),
      in_specs=[
          pl.BlockSpec((blk, D_FEAT), lambda i: (i, 0)),
          pl.BlockSpec((blk, D_FEAT), lambda i: (i, 0)),
      ],
      out_specs=pl.BlockSpec((blk, D_FEAT), lambda i: (i, 0)),
  )(a, b)


@jax.jit
def kernel(x, edge_index):
  src3 = edge_index[0].reshape(NW, NCHUNKS, CHUNK)
  partial = _sc_scatter_gather(x, src3, edge_index[1])
  return _tc_add(partial[0], partial[1])
